# baseline (speedup 1.0000x reference)
import functools

import jax
import jax.numpy as jnp
from jax.experimental import pallas as pl

_EPS = 1e-12
_BLOCK_ROWS = 512


def _ln_block(x_ref, g_ref, b_ref, o_ref):
    y = x_ref[...] + x_ref[...]
    mean = jnp.mean(y, axis=-1, keepdims=True)
    c = y - mean
    var = jnp.mean(c * c, axis=-1, keepdims=True)
    normed = c * jax.lax.rsqrt(var + _EPS)
    o_ref[...] = normed * g_ref[...] + b_ref[...]


@functools.partial(jax.jit, static_argnames=())
def kernel(x, lengths, gamma, beta):
    del lengths
    s, b, d = x.shape
    rows = s * b
    x2 = x.reshape(rows, d)
    br = _BLOCK_ROWS if rows % _BLOCK_ROWS == 0 else rows
    out = pl.pallas_call(
        _ln_block,
        grid=(rows // br,),
        in_specs=[
            pl.BlockSpec((br, d), lambda i: (i, 0)),
            pl.BlockSpec((1, d), lambda i: (0, 0)),
            pl.BlockSpec((1, d), lambda i: (0, 0)),
        ],
        out_specs=pl.BlockSpec((br, d), lambda i: (i, 0)),
        out_shape=jax.ShapeDtypeStruct((rows, d), x.dtype),
    )(x2, gamma.reshape(1, d), beta.reshape(1, d))
    return out.reshape(s, b, d)
